# Initial kernel scaffold; baseline (speedup 1.0000x reference)
import jax
import jax.numpy as jnp
from jax.experimental import pallas as pl


def kernel(atomic_numbers, positions, idx_i, idx_j, atom_features):
    raise NotImplementedError("write your pallas kernel here")


# speedup vs baseline: 3.1525x; 3.1525x over previous
import functools

import jax
import jax.numpy as jnp
import numpy as np
from jax import lax
from jax.experimental import pallas as pl
from jax.experimental.pallas import tpu as pltpu
from jax.experimental.pallas import tpu_sc as plsc

N_ATOMS = 100000
N_EDGES = 1600000
N_FEAT = 128
K_RBF = 32
RC = 8.0

NC = 2
NS = 16
NW = NC * NS

NA_PAD = 102400
AB = 1024
AB_ROWS = AB // 128
ABH = AB // 2
A_BLKS = NA_PAD // AB
A_ITERS = -(-A_BLKS // NW)

NE_PAD = 1638400
EB = 2048
EB_ROWS = EB // 128
G16 = EB // 16
E_BLKS = NE_PAD // EB
E_ITERS = E_BLKS // NW

_mesh = plsc.VectorSubcoreMesh(core_axis_name="c", subcore_axis_name="s")


@functools.partial(
    pl.kernel,
    mesh=_mesh,
    compiler_params=pltpu.CompilerParams(needs_layout_passes=False),
    out_type=(
        jax.ShapeDtypeStruct((NA_PAD, N_FEAT), jnp.float32),
        jax.ShapeDtypeStruct((NE_PAD,), jnp.float32),
    ),
    scratch_types=[
        pltpu.VMEM((AB_ROWS, 128), jnp.int32),
        pltpu.VMEM((128, N_FEAT), jnp.float32),
        pltpu.VMEM((N_ATOMS,), jnp.float32),
        pltpu.VMEM((EB,), jnp.int32),
        pltpu.VMEM((EB,), jnp.int32),
        pltpu.VMEM((EB,), jnp.float32),
        pltpu.SemaphoreType.DMA,
    ],
)
def _sc_gather(feat_hbm, an_hbm, px_hbm, py_hbm, pz_hbm, ii_hbm, jj_hbm,
               featout_hbm, d2_hbm,
               aidx_v, arows_v, ptab_v, iidx_v, jidx_v, d2_v, sem):
    wid = lax.axis_index("s") * NC + lax.axis_index("c")

    def atom_block(n, carry):
        t = n * NW + wid

        @pl.when(t < A_BLKS)
        def _():
            pltpu.sync_copy(an_hbm.at[t], aidx_v)
            for j in range(AB_ROWS):
                pltpu.async_copy(feat_hbm.at[aidx_v.at[j]],
                                 arows_v, sem).wait()
                pltpu.sync_copy(
                    arows_v, featout_hbm.at[pl.ds(t * AB + j * 128, 128)])

        return carry

    lax.fori_loop(0, A_ITERS, atom_block, 0)

    for c, ptab_hbm in enumerate((px_hbm, py_hbm, pz_hbm)):
        pltpu.sync_copy(ptab_hbm, ptab_v)

        def edge_block(n, carry):
            t = n * NW + wid
            base = t * EB
            pltpu.sync_copy(ii_hbm.at[pl.ds(base, EB)], iidx_v)
            pltpu.sync_copy(jj_hbm.at[pl.ds(base, EB)], jidx_v)
            if c > 0:
                pltpu.sync_copy(d2_hbm.at[pl.ds(base, EB)], d2_v)

            def grp(g, cc):
                sl = pl.ds(g * 16, 16)
                xi = plsc.load_gather(ptab_v, [iidx_v[sl]])
                xj = plsc.load_gather(ptab_v, [jidx_v[sl]])
                dx = xj - xi
                if c > 0:
                    d2_v[sl] = d2_v[sl] + dx * dx
                else:
                    d2_v[sl] = dx * dx
                return cc

            lax.fori_loop(0, G16, grp, 0)
            pltpu.sync_copy(d2_v, d2_hbm.at[pl.ds(base, EB)])
            return carry

        lax.fori_loop(0, E_ITERS, edge_block, 0)


_BE = 12800
_GE = N_EDGES // _BE
_WIDTH = float(((2.0 / K_RBF) * (np.exp(-0.0) - np.exp(-RC))) ** (-2))


def _rbf_body(d2_ref, cen_ref, dist_ref, rbf_ref):
    d2 = d2_ref[...]
    d = jnp.sqrt(d2 + 1e-12)
    dist_ref[...] = d
    x = d * (1.0 / RC)
    x2 = x * x
    x3 = x2 * x
    x4 = x2 * x2
    x5 = x4 * x
    fc = 1.0 - 6.0 * x5 + 15.0 * x4 - 10.0 * x3
    fc = jnp.where(d < RC, fc, 0.0)
    t = jnp.exp(-d)
    ones = jnp.ones((1, K_RBF), jnp.float32)
    dims = (((0,), (0,)), ((), ()))
    tb = lax.dot_general(t, ones, dims, precision=lax.Precision.HIGHEST,
                         preferred_element_type=jnp.float32)
    fb = lax.dot_general(fc, ones, dims, precision=lax.Precision.HIGHEST,
                         preferred_element_type=jnp.float32)
    diff = tb - cen_ref[...]
    rbf_ref[...] = fb * jnp.exp((-_WIDTH) * (diff * diff))


_rbf_call = pl.pallas_call(
    _rbf_body,
    grid=(_GE,),
    in_specs=[
        pl.BlockSpec((1, _BE), lambda i: (0, i)),
        pl.BlockSpec((1, K_RBF), lambda i: (0, 0)),
    ],
    out_specs=[
        pl.BlockSpec((1, _BE), lambda i: (0, i)),
        pl.BlockSpec((_BE, K_RBF), lambda i: (i, 0)),
    ],
    out_shape=[
        jax.ShapeDtypeStruct((1, N_EDGES), jnp.float32),
        jax.ShapeDtypeStruct((N_EDGES, K_RBF), jnp.float32),
    ],
)


def kernel(atomic_numbers, positions, idx_i, idx_j, atom_features):
    an = jnp.pad(atomic_numbers.astype(jnp.int32),
                 (0, NA_PAD - N_ATOMS)).reshape(A_BLKS, AB_ROWS, 128)
    ii = jnp.pad(idx_i.astype(jnp.int32), (0, NE_PAD - N_EDGES))
    jj = jnp.pad(idx_j.astype(jnp.int32), (0, NE_PAD - N_EDGES))
    pos = positions.astype(jnp.float32)
    px, py, pz = pos[:, 0], pos[:, 1], pos[:, 2]
    feat = atom_features.astype(jnp.float32)

    feats_p, d2_p = _sc_gather(feat, an, px, py, pz, ii, jj)
    features = feats_p[:N_ATOMS]

    d2 = d2_p[:N_EDGES].reshape(1, N_EDGES)
    centers = jnp.asarray(
        np.linspace(np.exp(-0.0), np.exp(-RC), K_RBF),
        jnp.float32).reshape(1, K_RBF)
    dist2d, rbfs = _rbf_call(d2, centers)
    distances = dist2d.reshape(-1)
    return features, rbfs, distances


# speedup vs baseline: 5.1705x; 1.6401x over previous
import functools

import jax
import jax.numpy as jnp
import numpy as np
from jax import lax
from jax.experimental import pallas as pl
from jax.experimental.pallas import tpu as pltpu
from jax.experimental.pallas import tpu_sc as plsc

N_ATOMS = 100000
N_EDGES = 1600000
N_FEAT = 128
K_RBF = 32
RC = 8.0

NC = 2
NS = 16
NW = NC * NS

NA_PAD = 102400
AB = 1024
AB_ROWS = AB // 128
ABH = AB // 2
A_BLKS = NA_PAD // AB
A_ITERS = -(-A_BLKS // NW)

NE_PAD = 1638400
EB = 2048
EB_ROWS = EB // 128
G16 = EB // 16
E_BLKS = NE_PAD // EB
E_ITERS = E_BLKS // NW

_mesh = plsc.VectorSubcoreMesh(core_axis_name="c", subcore_axis_name="s")


@functools.partial(
    pl.kernel,
    mesh=_mesh,
    compiler_params=pltpu.CompilerParams(needs_layout_passes=False),
    out_type=(
        jax.ShapeDtypeStruct((NA_PAD, N_FEAT), jnp.float32),
        jax.ShapeDtypeStruct((NE_PAD,), jnp.float32),
    ),
    scratch_types=[
        pltpu.SemaphoreType.DMA,
    ],
)
def _sc_gather(feat_hbm, an_hbm, px_hbm, py_hbm, pz_hbm, ii_hbm, jj_hbm,
               featout_hbm, d2_hbm, sem):
    wid = lax.axis_index("s") * NC + lax.axis_index("c")

    def atom_phase(aidx_v, arows_v):
        def atom_block(n, carry):
            t = n * NW + wid

            @pl.when(t < A_BLKS)
            def _():
                pltpu.sync_copy(an_hbm.at[t], aidx_v)
                for h in range(2):
                    handles = [
                        pltpu.async_copy(
                            feat_hbm.at[aidx_v.at[h * 4 + j]],
                            arows_v.at[pl.ds(j * 128, 128)], sem)
                        for j in range(4)
                    ]
                    for hd in handles:
                        hd.wait()
                    pltpu.sync_copy(
                        arows_v,
                        featout_hbm.at[pl.ds(t * AB + h * ABH, ABH)])

            return carry

        lax.fori_loop(0, A_ITERS, atom_block, 0)

    pl.run_scoped(atom_phase,
                  pltpu.VMEM((AB_ROWS, 128), jnp.int32),
                  pltpu.VMEM((ABH, N_FEAT), jnp.float32))

    def edge_phase(ptab_v, iidx_v, jidx_v, d2_v):
        for c, ptab_hbm in enumerate((px_hbm, py_hbm, pz_hbm)):
            pltpu.sync_copy(ptab_hbm, ptab_v)

            def edge_block(n, carry):
                t = n * NW + wid
                base = t * EB
                handles = [
                    pltpu.async_copy(ii_hbm.at[pl.ds(base, EB)], iidx_v, sem),
                    pltpu.async_copy(jj_hbm.at[pl.ds(base, EB)], jidx_v, sem),
                ]
                if c > 0:
                    handles.append(pltpu.async_copy(
                        d2_hbm.at[pl.ds(base, EB)], d2_v, sem))
                for hd in handles:
                    hd.wait()

                @plsc.parallel_loop(0, EB, 16, unroll=8)
                def grp(i):
                    sl = pl.ds(i, 16)
                    xi = plsc.load_gather(ptab_v, [iidx_v[sl]])
                    xj = plsc.load_gather(ptab_v, [jidx_v[sl]])
                    dx = xj - xi
                    if c > 0:
                        d2_v[sl] = d2_v[sl] + dx * dx
                    else:
                        d2_v[sl] = dx * dx

                pltpu.sync_copy(d2_v, d2_hbm.at[pl.ds(base, EB)])
                return carry

            lax.fori_loop(0, E_ITERS, edge_block, 0)

    pl.run_scoped(edge_phase,
                  pltpu.VMEM((N_ATOMS,), jnp.float32),
                  pltpu.VMEM((EB,), jnp.int32),
                  pltpu.VMEM((EB,), jnp.int32),
                  pltpu.VMEM((EB,), jnp.float32))


_BE = 12800
_GE = N_EDGES // _BE
_WIDTH = float(((2.0 / K_RBF) * (np.exp(-0.0) - np.exp(-RC))) ** (-2))


def _rbf_body(d2_ref, cen_ref, dist_ref, rbf_ref):
    d2 = d2_ref[...]
    d = jnp.sqrt(d2 + 1e-12)
    dist_ref[...] = d
    x = d * (1.0 / RC)
    x2 = x * x
    x3 = x2 * x
    x4 = x2 * x2
    x5 = x4 * x
    fc = 1.0 - 6.0 * x5 + 15.0 * x4 - 10.0 * x3
    fc = jnp.where(d < RC, fc, 0.0)
    t = jnp.exp(-d)
    th = t.astype(jnp.bfloat16).astype(jnp.float32)
    tl = t - th
    ones = jnp.ones((1, K_RBF), jnp.float32)
    dims = (((0,), (0,)), ((), ()))
    tb = (lax.dot_general(th, ones, dims, preferred_element_type=jnp.float32)
          + lax.dot_general(tl, ones, dims, preferred_element_type=jnp.float32))
    fb = lax.dot_general(fc, ones, dims, preferred_element_type=jnp.float32)
    diff = tb - cen_ref[...]
    rbf_ref[...] = fb * jnp.exp((-_WIDTH) * (diff * diff))


_rbf_call = pl.pallas_call(
    _rbf_body,
    grid=(_GE,),
    in_specs=[
        pl.BlockSpec((1, _BE), lambda i: (0, i)),
        pl.BlockSpec((1, K_RBF), lambda i: (0, 0)),
    ],
    out_specs=[
        pl.BlockSpec((1, _BE), lambda i: (0, i)),
        pl.BlockSpec((_BE, K_RBF), lambda i: (i, 0)),
    ],
    out_shape=[
        jax.ShapeDtypeStruct((1, N_EDGES), jnp.float32),
        jax.ShapeDtypeStruct((N_EDGES, K_RBF), jnp.float32),
    ],
)


def kernel(atomic_numbers, positions, idx_i, idx_j, atom_features):
    an = jnp.pad(atomic_numbers.astype(jnp.int32),
                 (0, NA_PAD - N_ATOMS)).reshape(A_BLKS, AB_ROWS, 128)
    ii = jnp.pad(idx_i.astype(jnp.int32), (0, NE_PAD - N_EDGES))
    jj = jnp.pad(idx_j.astype(jnp.int32), (0, NE_PAD - N_EDGES))
    pos = positions.astype(jnp.float32)
    px, py, pz = pos[:, 0], pos[:, 1], pos[:, 2]
    feat = atom_features.astype(jnp.float32)

    feats_p, d2_p = _sc_gather(feat, an, px, py, pz, ii, jj)
    features = feats_p[:N_ATOMS]

    d2 = d2_p[:N_EDGES].reshape(1, N_EDGES)
    centers = jnp.asarray(
        np.linspace(np.exp(-0.0), np.exp(-RC), K_RBF),
        jnp.float32).reshape(1, K_RBF)
    dist2d, rbfs = _rbf_call(d2, centers)
    distances = dist2d.reshape(-1)
    return features, rbfs, distances


# speedup vs baseline: 6.2922x; 1.2169x over previous
import functools

import jax
import jax.numpy as jnp
import numpy as np
from jax import lax
from jax.experimental import pallas as pl
from jax.experimental.pallas import tpu as pltpu
from jax.experimental.pallas import tpu_sc as plsc

N_ATOMS = 100000
N_EDGES = 1600000
N_FEAT = 128
K_RBF = 32
RC = 8.0

NC = 2
NS = 16
NW = NC * NS

NA_PAD = 102400
AB = 1024
AB_ROWS = AB // 128
ABH = AB // 2
A_BLKS = NA_PAD // AB
A_ITERS = -(-A_BLKS // NW)

NE_PAD = 1638400
EB = 3200
E_BLKS = NE_PAD // EB
E_ITERS = E_BLKS // NW

_mesh = plsc.VectorSubcoreMesh(core_axis_name="c", subcore_axis_name="s")


@functools.partial(
    pl.kernel,
    mesh=_mesh,
    compiler_params=pltpu.CompilerParams(needs_layout_passes=False),
    out_type=jax.ShapeDtypeStruct((NA_PAD, N_FEAT), jnp.float32),
    scratch_types=[
        pltpu.SemaphoreType.DMA,
    ],
)
def _sc_feats(feat_hbm, an_hbm, featout_hbm, sem):
    wid = lax.axis_index("s") * NC + lax.axis_index("c")

    def atom_phase(aidx_v, arows_v):
        def atom_block(n, carry):
            t = n * NW + wid

            @pl.when(t < A_BLKS)
            def _():
                pltpu.sync_copy(an_hbm.at[t], aidx_v)
                for h in range(2):
                    handles = [
                        pltpu.async_copy(
                            feat_hbm.at[aidx_v.at[h * 4 + j]],
                            arows_v.at[pl.ds(j * 128, 128)], sem)
                        for j in range(4)
                    ]
                    for hd in handles:
                        hd.wait()
                    pltpu.sync_copy(
                        arows_v,
                        featout_hbm.at[pl.ds(t * AB + h * ABH, ABH)])

            return carry

        lax.fori_loop(0, A_ITERS, atom_block, 0)

    pl.run_scoped(atom_phase,
                  pltpu.VMEM((AB_ROWS, 128), jnp.int32),
                  pltpu.VMEM((ABH, N_FEAT), jnp.float32))


@functools.partial(
    pl.kernel,
    mesh=_mesh,
    compiler_params=pltpu.CompilerParams(needs_layout_passes=False),
    out_type=jax.ShapeDtypeStruct((NE_PAD,), jnp.float32),
    scratch_types=[
        pltpu.SemaphoreType.DMA,
    ],
)
def _sc_edges(px_hbm, py_hbm, pz_hbm, ii_hbm, jj_hbm, d2_hbm, sem):
    wid = lax.axis_index("s") * NC + lax.axis_index("c")

    def edge_phase(ptab_v, iidx_v, jidx_v, d2_v):
        for c, ptab_hbm in enumerate((px_hbm, py_hbm, pz_hbm)):
            pltpu.sync_copy(ptab_hbm, ptab_v)

            def edge_block(n, carry):
                t = n * NW + wid
                base = t * EB
                handles = [
                    pltpu.async_copy(ii_hbm.at[pl.ds(base, EB)], iidx_v, sem),
                    pltpu.async_copy(jj_hbm.at[pl.ds(base, EB)], jidx_v, sem),
                ]
                if c > 0:
                    handles.append(pltpu.async_copy(
                        d2_hbm.at[pl.ds(base, EB)], d2_v, sem))
                for hd in handles:
                    hd.wait()

                @plsc.parallel_loop(0, EB, 16, unroll=8)
                def grp(i):
                    sl = pl.ds(i, 16)
                    xi = plsc.load_gather(ptab_v, [iidx_v[sl]])
                    xj = plsc.load_gather(ptab_v, [jidx_v[sl]])
                    dx = xj - xi
                    if c > 0:
                        d2_v[sl] = d2_v[sl] + dx * dx
                    else:
                        d2_v[sl] = dx * dx

                pltpu.sync_copy(d2_v, d2_hbm.at[pl.ds(base, EB)])
                return carry

            lax.fori_loop(0, E_ITERS, edge_block, 0)

    pl.run_scoped(edge_phase,
                  pltpu.VMEM((N_ATOMS,), jnp.float32),
                  pltpu.VMEM((EB,), jnp.int32),
                  pltpu.VMEM((EB,), jnp.int32),
                  pltpu.VMEM((EB,), jnp.float32))


_BE = 12800
_GE = N_EDGES // _BE
_WIDTH = float(((2.0 / K_RBF) * (np.exp(-0.0) - np.exp(-RC))) ** (-2))


def _rbf_body(d2_ref, cen_ref, dist_ref, rbf_ref):
    d2 = d2_ref[...]
    d = jnp.sqrt(d2 + 1e-12)
    dist_ref[...] = d
    x = d * (1.0 / RC)
    x2 = x * x
    x3 = x2 * x
    x4 = x2 * x2
    x5 = x4 * x
    fc = 1.0 - 6.0 * x5 + 15.0 * x4 - 10.0 * x3
    fc = jnp.where(d < RC, fc, 0.0)
    t = jnp.exp(-d)
    th = t.astype(jnp.bfloat16).astype(jnp.float32)
    tl = t - th
    ones = jnp.ones((1, K_RBF), jnp.float32)
    dims = (((0,), (0,)), ((), ()))
    tb = (lax.dot_general(th, ones, dims, preferred_element_type=jnp.float32)
          + lax.dot_general(tl, ones, dims, preferred_element_type=jnp.float32))
    fb = lax.dot_general(fc, ones, dims, preferred_element_type=jnp.float32)
    diff = tb - cen_ref[...]
    rbf_ref[...] = fb * jnp.exp((-_WIDTH) * (diff * diff))


_rbf_call = pl.pallas_call(
    _rbf_body,
    grid=(_GE,),
    in_specs=[
        pl.BlockSpec((1, _BE), lambda i: (0, i)),
        pl.BlockSpec((1, K_RBF), lambda i: (0, 0)),
    ],
    out_specs=[
        pl.BlockSpec((1, _BE), lambda i: (0, i)),
        pl.BlockSpec((_BE, K_RBF), lambda i: (i, 0)),
    ],
    out_shape=[
        jax.ShapeDtypeStruct((1, N_EDGES), jnp.float32),
        jax.ShapeDtypeStruct((N_EDGES, K_RBF), jnp.float32),
    ],
)


def kernel(atomic_numbers, positions, idx_i, idx_j, atom_features):
    an = jnp.pad(atomic_numbers.astype(jnp.int32),
                 (0, NA_PAD - N_ATOMS)).reshape(A_BLKS, AB_ROWS, 128)
    ii = jnp.pad(idx_i.astype(jnp.int32), (0, NE_PAD - N_EDGES))
    jj = jnp.pad(idx_j.astype(jnp.int32), (0, NE_PAD - N_EDGES))
    pos = positions.astype(jnp.float32)
    px, py, pz = pos[:, 0], pos[:, 1], pos[:, 2]
    feat = atom_features.astype(jnp.float32)

    d2_p = _sc_edges(px, py, pz, ii, jj)
    feats_p = _sc_feats(feat, an)
    features = feats_p[:N_ATOMS]

    d2 = d2_p[:N_EDGES].reshape(1, N_EDGES)
    centers = jnp.asarray(
        np.linspace(np.exp(-0.0), np.exp(-RC), K_RBF),
        jnp.float32).reshape(1, K_RBF)
    dist2d, rbfs = _rbf_call(d2, centers)
    distances = dist2d.reshape(-1)
    return features, rbfs, distances
